# Initial kernel scaffold; baseline (speedup 1.0000x reference)
import jax
import jax.numpy as jnp
from jax.experimental import pallas as pl


def kernel(x, w_gate, W1, W2):
    raise NotImplementedError("write your pallas kernel here")


# speedup vs baseline: 1.0895x; 1.0895x over previous
import functools
import jax
import jax.numpy as jnp
from jax.experimental import pallas as pl
from jax.experimental.pallas import tpu as pltpu

_N, _D, _H, _E, _TOPK = 4096, 1024, 1024, 8, 2
_TB = 256
_NTB = _N // _TB
_BLK = 256
_NB = (_N * _TOPK) // _BLK + _E
_P = _NB * _BLK


def _router_body(x_ref, wg_ref, e0_ref, e1_ref, g0_ref, g1_ref,
                 r0_ref, r1_ref, cnt_ref, cnt_acc):
    i = pl.program_id(0)

    @pl.when(i == 0)
    def _():
        cnt_acc[...] = jnp.zeros_like(cnt_acc)

    logits = jnp.dot(x_ref[...], wg_ref[...],
                     preferred_element_type=jnp.float32)
    colsi = jax.lax.broadcasted_iota(jnp.int32, (_TB, _E), 1)
    m0 = jnp.max(logits, axis=1, keepdims=True)
    e0 = jnp.min(jnp.where(logits == m0, colsi, _E), axis=1, keepdims=True)
    oh0 = (colsi == e0).astype(jnp.float32)
    l1 = jnp.where(colsi == e0, -1e30, logits)
    m1 = jnp.max(l1, axis=1, keepdims=True)
    e1 = jnp.min(jnp.where(l1 == m1, colsi, _E), axis=1, keepdims=True)
    oh1 = (colsi == e1).astype(jnp.float32)
    g0 = 1.0 / (1.0 + jnp.exp(m1 - m0))
    g1 = 1.0 - g0
    rows = jax.lax.broadcasted_iota(jnp.int32, (_TB, _TB), 0)
    cols = jax.lax.broadcasted_iota(jnp.int32, (_TB, _TB), 1)
    lt = (cols < rows).astype(jnp.float32)
    cum0 = jnp.dot(lt, oh0, preferred_element_type=jnp.float32)
    cum1 = jnp.dot(lt, oh1, preferred_element_type=jnp.float32)
    cnt = cnt_acc[...]
    tot0 = jnp.sum(oh0, axis=0, keepdims=True)
    tot1 = jnp.sum(oh1, axis=0, keepdims=True)
    r0 = jnp.sum(oh0 * (cnt + cum0), axis=1, keepdims=True)
    r1 = jnp.sum(oh1 * (cnt + tot0 + cum1), axis=1, keepdims=True)
    new_cnt = cnt + tot0 + tot1
    cnt_acc[...] = new_cnt
    cnt_ref[...] = new_cnt
    e0_ref[...] = e0
    e1_ref[...] = e1
    g0_ref[...] = g0
    g1_ref[...] = g1
    r0_ref[...] = r0.astype(jnp.int32)
    r1_ref[...] = r1.astype(jnp.int32)


def _router(x, w_gate):
    col = lambda dt: jax.ShapeDtypeStruct((_N, 1), dt)
    out_shapes = (col(jnp.int32), col(jnp.int32), col(jnp.float32),
                  col(jnp.float32), col(jnp.int32), col(jnp.int32),
                  jax.ShapeDtypeStruct((1, _E), jnp.float32))
    colspec = lambda: pl.BlockSpec((_TB, 1), lambda i: (i, 0))
    return pl.pallas_call(
        _router_body,
        grid=(_NTB,),
        in_specs=[
            pl.BlockSpec((_TB, _D), lambda i: (i, 0)),
            pl.BlockSpec((_D, _E), lambda i: (0, 0)),
        ],
        out_specs=(colspec(), colspec(), colspec(), colspec(),
                   colspec(), colspec(),
                   pl.BlockSpec((1, _E), lambda i: (0, 0))),
        out_shape=out_shapes,
        scratch_shapes=[pltpu.VMEM((1, _E), jnp.float32)],
    )(x, w_gate)


def _offsets_body(cnt_ref, off_ref, blk_ref):
    c = jnp.round(cnt_ref[...]).astype(jnp.int32)
    nb = (c + (_BLK - 1)) >> 8
    cpad = (nb << 8).astype(jnp.float32)
    f = jax.lax.broadcasted_iota(jnp.int32, (_E, _E), 0)
    e = jax.lax.broadcasted_iota(jnp.int32, (_E, _E), 1)
    ut = (f < e).astype(jnp.float32)
    off = jnp.dot(cpad, ut, preferred_element_type=jnp.float32)
    off_ref[...] = jnp.round(off).astype(jnp.int32)
    ib = jax.lax.broadcasted_iota(jnp.int32, (_NB, _E), 0) * _BLK
    le = (off.astype(jnp.int32) <= ib).astype(jnp.int32)
    blk = jnp.sum(le, axis=1, keepdims=True) - 1
    blk_ref[...] = jnp.clip(blk, 0, _E - 1)


def _offsets(cnt):
    return pl.pallas_call(
        _offsets_body,
        in_specs=[pl.BlockSpec((1, _E), lambda: (0, 0))],
        out_specs=(pl.BlockSpec((1, _E), lambda: (0, 0)),
                   pl.BlockSpec((_NB, 1), lambda: (0, 0))),
        out_shape=(jax.ShapeDtypeStruct((1, _E), jnp.int32),
                   jax.ShapeDtypeStruct((_NB, 1), jnp.int32)),
    )(cnt)


def _gmm_body(m_ref, xs_ref, w1_ref, w2_ref, ys_ref):
    h = jnp.maximum(
        jnp.dot(xs_ref[...], w1_ref[0], preferred_element_type=jnp.float32),
        0.0)
    ys_ref[...] = jnp.dot(h, w2_ref[0], preferred_element_type=jnp.float32)


def _grouped_matmul(xs, W1, W2, blk_expert):
    grid_spec = pltpu.PrefetchScalarGridSpec(
        num_scalar_prefetch=1,
        grid=(_NB,),
        in_specs=[
            pl.BlockSpec((_BLK, _D), lambda i, m: (i, 0)),
            pl.BlockSpec((1, _D, _H), lambda i, m: (m[i], 0, 0)),
            pl.BlockSpec((1, _H, _D), lambda i, m: (m[i], 0, 0)),
        ],
        out_specs=pl.BlockSpec((_BLK, _D), lambda i, m: (i, 0)),
    )
    return pl.pallas_call(
        _gmm_body,
        grid_spec=grid_spec,
        out_shape=jax.ShapeDtypeStruct((_P, _D), jnp.float32),
    )(blk_expert, xs, W1, W2)


def kernel(x, w_gate, W1, W2):
    e0, e1, g0, g1, r0, r1, cnt = _router(x, w_gate)
    off, blk_expert = _offsets(cnt)
    off = off.reshape(_E)
    e0, e1, r0, r1 = (a.reshape(_N) for a in (e0, e1, r0, r1))
    pos0 = off[e0] + r0
    pos1 = off[e1] + r1
    xs = jnp.zeros((_P, _D), jnp.float32)
    xs = xs.at[pos0].set(x * g0).at[pos1].set(x * g1)
    ys = _grouped_matmul(xs, W1, W2, blk_expert.reshape(_NB))
    return ys[pos0] + ys[pos1]
